# baseline (speedup 1.0000x reference)
import functools

import jax
import jax.numpy as jnp
from jax import lax
from jax.experimental import pallas as pl
from jax.experimental.pallas import tpu as pltpu
from jax.experimental.pallas import tpu_sc as plsc

B = 4096
L = 200
C = 64
LT = L // 8
NCH = L // 2

_info = plsc.get_sparse_core_info()
NC = _info.num_cores
NS = _info.num_subcores
NW = NC * NS
NBB = B // NW
NP = NCH // 2


def _body(xt_hbm, w_hbm, out_hbm, idx_v, rows_v, outc_v, gsem, osem):
    wid = lax.axis_index("s") * NC + lax.axis_index("c")

    iota = lax.iota(jnp.int32, 16)
    cidx = [iota + cb * 16 for cb in range(4)]

    def start_gather(slot, ch):
        for h in range(2):
            pltpu.async_copy(
                w_hbm.at[idx_v.at[ch, h]], rows_v.at[slot, h], gsem.at[slot])

    def wait_gather(slot):
        for h in range(2):
            pltpu.make_async_copy(
                w_hbm.at[idx_v.at[0, 0]], rows_v.at[slot, h],
                gsem.at[slot]).wait()

    def start_out(slot, ch):
        lt = ch // 4
        ls = (ch % 4) * 2
        for h in range(2):
            pltpu.async_copy(
                outc_v.at[slot, h], out_hbm.at[:, lt, wid, ls + h],
                osem.at[slot])

    def wait_out(slot):
        for h in range(2):
            pltpu.make_async_copy(
                outc_v.at[slot, h], out_hbm.at[:, 0, 0, 0],
                osem.at[slot]).wait()

    def transpose_chunk(slot):
        for h in range(2):
            def per_rb(rb, c2, h=h):
                rbv = jnp.full((16,), rb, dtype=jnp.int32)
                for cb in range(4):
                    v = rows_v[slot, h, rb, pl.ds(cb * 16, 16)]
                    plsc.store_scatter(outc_v.at[slot, h], [cidx[cb], rbv], v)
                return c2
            lax.fori_loop(0, NBB, per_rb, 0)

    pltpu.sync_copy(xt_hbm.at[:, :, wid], idx_v)
    start_gather(0, 0)

    def per_pair(p, carry):
        ch0 = 2 * p
        start_gather(1, ch0 + 1)
        wait_gather(0)
        @pl.when(p > 0)
        def _():
            wait_out(0)
        transpose_chunk(0)
        start_out(0, ch0)
        @pl.when(p < NP - 1)
        def _():
            start_gather(0, ch0 + 2)
        wait_gather(1)
        @pl.when(p > 0)
        def _():
            wait_out(1)
        transpose_chunk(1)
        start_out(1, ch0 + 1)
        return carry

    lax.fori_loop(0, NP, per_pair, 0)
    wait_out(0)
    wait_out(1)


@functools.partial(jax.jit, static_argnames=())
def _sc_embed(xt, w):
    mesh = plsc.VectorSubcoreMesh(core_axis_name="c", subcore_axis_name="s")
    f = pl.kernel(
        _body,
        mesh=mesh,
        out_type=jax.ShapeDtypeStruct((C, LT, NW, 8, NBB), jnp.float32),
        scratch_types=[
            pltpu.VMEM((NCH, 2, NBB), jnp.int32),
            pltpu.VMEM((2, 2, NBB, C), jnp.float32),
            pltpu.VMEM((2, 2, C, NBB), jnp.float32),
            pltpu.SemaphoreType.DMA((2,)),
            pltpu.SemaphoreType.DMA((2,)),
        ],
        compiler_params=pltpu.CompilerParams(
            needs_layout_passes=False, use_tc_tiling_on_sc=False),
    )
    return f(xt, w)


def kernel(x, W):
    xt = jnp.transpose(x).astype(jnp.int32).reshape(NCH, 2, NW, NBB)
    res = _sc_embed(xt, W)
    return res.transpose(2, 4, 0, 1, 3).reshape(B, C, L)
